# Initial kernel scaffold; baseline (speedup 1.0000x reference)
import jax
import jax.numpy as jnp
import numpy as np
from jax import lax
from jax.experimental import pallas as pl
from jax.experimental.pallas import tpu as pltpu
from jax.experimental.pallas import tpu_sc as plsc

N = 10000
E = 320000
DIM = 128
H = 8
HD = DIM // H
NC = 2
NS = 16
NW = NC * NS
EPW = E // NW
CHUNK = 80
NCHUNK = EPW // CHUNK
NPT = N // NS
ACC_W = 144


def _qkv_body(x_ref, w_ref, b_ref, y_ref):
    y_ref[...] = (
        jnp.dot(x_ref[...], w_ref[...], preferred_element_type=jnp.float32)
        + b_ref[...]
    )


def _edge_body(q_hbm, k_hbm, v_hbm, src_hbm, dst_hbm, z_hbm, out_hbm,
               acc, src_v, dst_v, qb, kb, vb, wrow, sq, sk, sv):
    cid = lax.axis_index("c")
    sid = lax.axis_index("s")
    wid = cid * NS + sid

    pltpu.sync_copy(z_hbm.at[pl.ds(sid * NPT, NPT)],
                    acc.at[pl.ds(sid * NPT, NPT)])
    plsc.subcore_barrier()

    ebase = wid * EPW
    lanes = lax.iota(jnp.int32, HD)

    def chunk_body(c, carry):
        off = ebase + c * CHUNK
        pltpu.sync_copy(src_hbm.at[pl.ds(off, CHUNK)], src_v)
        pltpu.sync_copy(dst_hbm.at[pl.ds(off, CHUNK)], dst_v)
        cq = pltpu.async_copy(q_hbm.at[src_v], qb, sq)
        ck = pltpu.async_copy(k_hbm.at[dst_v], kb, sk)
        cv = pltpu.async_copy(v_hbm.at[src_v], vb, sv)
        cq.wait()
        ck.wait()
        cv.wait()

        def edge_body(j, ecarry):
            tail = jnp.zeros((HD,), jnp.float32)
            for h in range(H):
                qv = qb[j, pl.ds(h * HD, HD)]
                kv = kb[j, pl.ds(h * HD, HD)]
                s = jnp.sum(qv * kv) * 0.25
                w = jnp.exp(jnp.broadcast_to(s, (HD,)))
                wrow[j, pl.ds(h * HD, HD)] = vb[j, pl.ds(h * HD, HD)] * w
                tail = jnp.where(lanes == h, w, tail)
            wrow[j, pl.ds(DIM, HD)] = tail
            return ecarry

        lax.fori_loop(0, CHUNK, edge_body, 0)
        pltpu.sync_copy(wrow, acc.at[dst_v], add=True)
        return carry

    lax.fori_loop(0, NCHUNK, chunk_body, 0)
    plsc.subcore_barrier()
    pltpu.sync_copy(acc.at[pl.ds(sid * NPT, NPT)],
                    out_hbm.at[pl.ds(cid * N + sid * NPT, NPT)])


def _combine_body(p_ref, sel_ref, wo_ref, bo_ref, o_ref):
    psum = p_ref[0] + p_ref[1]
    num = psum[:, :DIM]
    den = jnp.dot(psum, sel_ref[...], preferred_element_type=jnp.float32)
    pos = den > 0.0
    attn = jnp.where(pos, num / jnp.where(pos, den, 1.0), 0.0)
    o_ref[...] = (
        jnp.dot(attn, wo_ref[...], preferred_element_type=jnp.float32)
        + bo_ref[...]
    )


_SEL = np.zeros((ACC_W, DIM), np.float32)
for _h in range(H):
    _SEL[DIM + _h, _h * HD:(_h + 1) * HD] = 1.0
_SEL = jnp.asarray(_SEL)


def kernel(x, edge_index, Wq, bq, Wk, bk, Wv, bv, Wo, bo):
    wcat = jnp.concatenate([Wq.T, Wk.T, Wv.T], axis=1)
    bcat = jnp.concatenate([bq, bk, bv]).reshape(1, 3 * DIM)

    y = pl.pallas_call(
        _qkv_body,
        grid=(10,),
        in_specs=[
            pl.BlockSpec((1000, DIM), lambda i: (i, 0)),
            pl.BlockSpec((DIM, 3 * DIM), lambda i: (0, 0)),
            pl.BlockSpec((1, 3 * DIM), lambda i: (0, 0)),
        ],
        out_specs=pl.BlockSpec((1000, 3 * DIM), lambda i: (i, 0)),
        out_shape=jax.ShapeDtypeStruct((N, 3 * DIM), jnp.float32),
    )(x, wcat, bcat)
    q = y[:, :DIM]
    k = y[:, DIM:2 * DIM]
    v = y[:, 2 * DIM:]

    src32 = edge_index[0].astype(jnp.int32)
    dst32 = edge_index[1].astype(jnp.int32)
    zeros = jnp.zeros((N, ACC_W), jnp.float32)

    mesh = plsc.VectorSubcoreMesh(core_axis_name="c", subcore_axis_name="s")
    edge_fn = pl.kernel(
        _edge_body,
        mesh=mesh,
        out_type=jax.ShapeDtypeStruct((NC * N, ACC_W), jnp.float32),
        scratch_types=[
            pltpu.VMEM_SHARED((N, ACC_W), jnp.float32),
            pltpu.VMEM((CHUNK,), jnp.int32),
            pltpu.VMEM((CHUNK,), jnp.int32),
            pltpu.VMEM((CHUNK, DIM), jnp.float32),
            pltpu.VMEM((CHUNK, DIM), jnp.float32),
            pltpu.VMEM((CHUNK, DIM), jnp.float32),
            pltpu.VMEM((CHUNK, ACC_W), jnp.float32),
            pltpu.SemaphoreType.DMA,
            pltpu.SemaphoreType.DMA,
            pltpu.SemaphoreType.DMA,
        ],
    )
    partials = edge_fn(q, k, v, src32, dst32, zeros)
    partials = partials.reshape(NC, N, ACC_W)

    out = pl.pallas_call(
        _combine_body,
        grid=(10,),
        in_specs=[
            pl.BlockSpec((NC, 1000, ACC_W), lambda i: (0, i, 0)),
            pl.BlockSpec((ACC_W, DIM), lambda i: (0, 0)),
            pl.BlockSpec((DIM, DIM), lambda i: (0, 0)),
            pl.BlockSpec((1, DIM), lambda i: (0, 0)),
        ],
        out_specs=pl.BlockSpec((1000, DIM), lambda i: (i, 0)),
        out_shape=jax.ShapeDtypeStruct((N, DIM), jnp.float32),
    )(partials, _SEL, Wo.T, bo.reshape(1, DIM))
    return out


# speedup vs baseline: 45.4630x; 45.4630x over previous
import jax
import jax.numpy as jnp
import numpy as np
from jax import lax
from jax.experimental import pallas as pl
from jax.experimental.pallas import tpu as pltpu
from jax.experimental.pallas import tpu_sc as plsc

N = 10000
E = 320000
DIM = 128
H = 8
HD = DIM // H
NC = 2
NS = 16
NW = NC * NS
EPW = E // NW
CHUNK = 80
GRP = CHUNK // HD
NCHUNK = EPW // CHUNK
N_PAD = 10000
NPT = 1000
DROWS = 640
DPT = DROWS // NS
WDEN = 88


def _qkv_body(x_ref, w_ref, b_ref, y_ref):
    y_ref[...] = (
        jnp.dot(x_ref[...], w_ref[...], preferred_element_type=jnp.float32)
        + b_ref[...]
    )


def _edge_body(q_hbm, k_hbm, v_hbm, src_hbm, dst_hbm, z_hbm,
               num_hbm, den_hbm,
               acc, dacc, src_v, dst_v, grow_v, qb, kb, vb, wden,
               sq, sk, sv):
    cid = lax.axis_index("c")
    sid = lax.axis_index("s")
    wid = cid * NS + sid

    @pl.when(sid < 10)
    def _init_num():
        pltpu.sync_copy(z_hbm.at[pl.ds(sid * NPT, NPT)],
                        acc.at[pl.ds(sid * NPT, NPT)])
    pltpu.sync_copy(z_hbm.at[pl.ds(sid * DPT, DPT)],
                    dacc.at[pl.ds(sid * DPT, DPT)])
    pltpu.sync_copy(z_hbm.at[pl.ds(0, WDEN)], wden)
    plsc.subcore_barrier()

    ebase = wid * EPW
    lanes = lax.iota(jnp.int32, HD)
    dnums = lax.GatherDimensionNumbers(
        offset_dims=(), collapsed_slice_dims=(0,), start_index_map=(0,))
    rot = [jnp.bitwise_xor(lanes, r).reshape(HD, 1) for r in (8, 4, 2, 1)]
    zeros16 = jnp.zeros((HD,), jnp.float32)

    def chunk_body(c, carry):
        off = ebase + c * CHUNK
        pltpu.sync_copy(src_hbm.at[pl.ds(off, CHUNK)], src_v)
        pltpu.sync_copy(dst_hbm.at[pl.ds(off, CHUNK)], dst_v)
        cq = pltpu.async_copy(q_hbm.at[src_v], qb, sq)
        ck = pltpu.async_copy(k_hbm.at[dst_v], kb, sk)
        cv = pltpu.async_copy(v_hbm.at[src_v], vb, sv)
        cq.wait()
        ck.wait()
        cv.wait()

        def group_body(g, gcarry):
            dgrp = dst_v[pl.ds(g * HD, HD)]
            grow_v[pl.ds(g * HD, HD)] = lax.shift_right_logical(dgrp, 4)
            for jj in range(HD):
                j = g * HD + jj
                tail = zeros16
                for h in range(H):
                    qv = qb[j, pl.ds(h * HD, HD)]
                    kv = kb[j, pl.ds(h * HD, HD)]
                    p = qv * kv
                    for idx in rot:
                        p = p + lax.gather(
                            p, idx, dnums, (1,),
                            mode=lax.GatherScatterMode.PROMISE_IN_BOUNDS)
                    w = jnp.exp(p * 0.25)
                    qb[j, pl.ds(h * HD, HD)] = vb[j, pl.ds(h * HD, HD)] * w
                    tail = jnp.where(lanes == h, w, tail)
                col = jnp.bitwise_and(dgrp[jj], 15) * H
                wden[j, pl.ds(col, HD)] = tail
            return gcarry

        lax.fori_loop(0, GRP, group_body, 0)
        pltpu.sync_copy(qb, acc.at[dst_v], add=True)
        pltpu.sync_copy(wden.at[pl.ds(0, CHUNK)], dacc.at[grow_v], add=True)

        def clean_body(g, gcarry):
            dgrp = dst_v[pl.ds(g * HD, HD)]
            for jj in range(HD):
                col = jnp.bitwise_and(dgrp[jj], 15) * H
                wden[g * HD + jj, pl.ds(col, HD)] = zeros16
            return gcarry

        lax.fori_loop(0, GRP, clean_body, 0)
        return carry

    lax.fori_loop(0, NCHUNK, chunk_body, 0)
    plsc.subcore_barrier()
    @pl.when(sid < 10)
    def _write_num():
        pltpu.sync_copy(acc.at[pl.ds(sid * NPT, NPT)],
                        num_hbm.at[pl.ds(cid * N_PAD + sid * NPT, NPT)])
    pltpu.sync_copy(dacc.at[pl.ds(sid * DPT, DPT)],
                    den_hbm.at[pl.ds(cid * DROWS + sid * DPT, DPT)])


def _combine_body(num_ref, den_ref, sel_ref, wo_ref, bo_ref, o_ref):
    num = num_ref[0] + num_ref[1]
    den8 = den_ref[0] + den_ref[1]
    den = jnp.dot(den8, sel_ref[...], preferred_element_type=jnp.float32)
    pos = den > 0.0
    attn = jnp.where(pos, num / jnp.where(pos, den, 1.0), 0.0)
    o_ref[...] = (
        jnp.dot(attn, wo_ref[...], preferred_element_type=jnp.float32)
        + bo_ref[...]
    )


_SEL = np.zeros((H, DIM), np.float32)
for _h in range(H):
    _SEL[_h, _h * HD:(_h + 1) * HD] = 1.0


def kernel(x, edge_index, Wq, bq, Wk, bk, Wv, bv, Wo, bo):
    wcat = jnp.concatenate([Wq.T, Wk.T, Wv.T], axis=1)
    bcat = jnp.concatenate([bq, bk, bv]).reshape(1, 3 * DIM)

    y = pl.pallas_call(
        _qkv_body,
        grid=(10,),
        in_specs=[
            pl.BlockSpec((1000, DIM), lambda i: (i, 0)),
            pl.BlockSpec((DIM, 3 * DIM), lambda i: (0, 0)),
            pl.BlockSpec((1, 3 * DIM), lambda i: (0, 0)),
        ],
        out_specs=pl.BlockSpec((1000, 3 * DIM), lambda i: (i, 0)),
        out_shape=jax.ShapeDtypeStruct((N, 3 * DIM), jnp.float32),
    )(x, wcat, bcat)
    q = y[:, :DIM]
    k = y[:, DIM:2 * DIM]
    v = y[:, 2 * DIM:]

    src32 = edge_index[0].astype(jnp.int32)
    dst32 = edge_index[1].astype(jnp.int32)
    zeros = jnp.zeros((N_PAD, DIM), jnp.float32)

    mesh = plsc.VectorSubcoreMesh(core_axis_name="c", subcore_axis_name="s")
    edge_fn = pl.kernel(
        _edge_body,
        mesh=mesh,
        out_type=[
            jax.ShapeDtypeStruct((NC * N_PAD, DIM), jnp.float32),
            jax.ShapeDtypeStruct((NC * DROWS, DIM), jnp.float32),
        ],
        scratch_types=[
            pltpu.VMEM_SHARED((N_PAD, DIM), jnp.float32),
            pltpu.VMEM_SHARED((DROWS, DIM), jnp.float32),
            pltpu.VMEM((CHUNK,), jnp.int32),
            pltpu.VMEM((CHUNK,), jnp.int32),
            pltpu.VMEM((CHUNK,), jnp.int32),
            pltpu.VMEM((CHUNK, DIM), jnp.float32),
            pltpu.VMEM((CHUNK, DIM), jnp.float32),
            pltpu.VMEM((CHUNK, DIM), jnp.float32),
            pltpu.VMEM((WDEN, DIM), jnp.float32),
            pltpu.SemaphoreType.DMA,
            pltpu.SemaphoreType.DMA,
            pltpu.SemaphoreType.DMA,
        ],
    )
    num, den = edge_fn(q, k, v, src32, dst32, zeros)
    num = num.reshape(NC, N_PAD, DIM)
    den = den.reshape(NC, DROWS * DIM // H, H)

    out = pl.pallas_call(
        _combine_body,
        grid=(10,),
        in_specs=[
            pl.BlockSpec((NC, 1000, DIM), lambda i: (0, i, 0)),
            pl.BlockSpec((NC, 1000, H), lambda i: (0, i, 0)),
            pl.BlockSpec((H, DIM), lambda i: (0, 0)),
            pl.BlockSpec((DIM, DIM), lambda i: (0, 0)),
            pl.BlockSpec((1, DIM), lambda i: (0, 0)),
        ],
        out_specs=pl.BlockSpec((1000, DIM), lambda i: (i, 0)),
        out_shape=jax.ShapeDtypeStruct((N, DIM), jnp.float32),
    )(num, den, jnp.asarray(_SEL), Wo.T, bo.reshape(1, DIM))
    return out
